# baseline (speedup 1.0000x reference)
import functools
import jax
import jax.numpy as jnp
from jax import lax
from jax.experimental import pallas as pl
from jax.experimental.pallas import tpu as pltpu

N = 50000
E = 800000
NG = 256
HID = 64
EDGE = 16
L = 3
NS = 4
MAX_Z = 118

NP_ = 50176
NBLK = 1792
NGRID = 28
EBLK = 4000
EGRID = E // EBLK

_f32 = jnp.float32


def _softplus(x):
    u = jnp.exp(-jnp.abs(x))
    zz = u / (2.0 + u)
    z2 = zz * zz
    p = 1.0 + z2 * (1.0 / 3.0 + z2 * (0.2 + z2 * (1.0 / 7.0)))
    return jnp.maximum(x, 0.0) + 2.0 * zz * p


def _x0_body(zc_ref, xs_ref, ae_ref, waet_ref, b0_ref, wxs_ref, wdt_ref, wst_ref,
             x0_ref, afs_ref, bfs_ref):
    zc = zc_ref[0, 0, :]
    t = jnp.dot(ae_ref[...], waet_ref[...], preferred_element_type=_f32, precision=lax.Precision.HIGHEST)
    oh = (zc[:, None] == lax.broadcasted_iota(jnp.int32, (NBLK, 128), 1)).astype(_f32)
    g = jnp.dot(oh, t, preferred_element_type=_f32, precision=lax.Precision.HIGHEST)
    x0 = g + jnp.dot(xs_ref[...], wxs_ref[...], preferred_element_type=_f32, precision=lax.Precision.HIGHEST) + b0_ref[...]
    x0_ref[...] = x0
    afs_ref[...] = jnp.dot(x0, wdt_ref[...], preferred_element_type=_f32, precision=lax.Precision.HIGHEST)
    bfs_ref[...] = jnp.dot(x0, wst_ref[...], preferred_element_type=_f32, precision=lax.Precision.HIGHEST)


def _x0_call(zc3, xs_p, ae_pad, waet, b0, wxs, wdt, wst):
    return pl.pallas_call(
        _x0_body,
        grid=(NGRID,),
        in_specs=[
            pl.BlockSpec((1, 1, NBLK), lambda i: (i, 0, 0)),
            pl.BlockSpec((NBLK, 8), lambda i: (i, 0)),
            pl.BlockSpec((128, 64), lambda i: (0, 0)),
            pl.BlockSpec((64, 64), lambda i: (0, 0)),
            pl.BlockSpec((1, 64), lambda i: (0, 0)),
            pl.BlockSpec((8, 64), lambda i: (0, 0)),
            pl.BlockSpec((64, 128), lambda i: (0, 0)),
            pl.BlockSpec((64, 128), lambda i: (0, 0)),
        ],
        out_specs=[
            pl.BlockSpec((NBLK, 64), lambda i: (i, 0)),
            pl.BlockSpec((NBLK, 128), lambda i: (i, 0)),
            pl.BlockSpec((NBLK, 128), lambda i: (i, 0)),
        ],
        out_shape=[
            jax.ShapeDtypeStruct((NP_, 64), _f32),
            jax.ShapeDtypeStruct((NP_, 128), _f32),
            jax.ShapeDtypeStruct((NP_, 128), _f32),
        ],
    )(zc3, xs_p, ae_pad, waet, b0, wxs, wdt, wst)


def _efs_body(ea_ref, w_ref, o_ref):
    o_ref[...] = jnp.dot(ea_ref[...], w_ref[...], preferred_element_type=_f32, precision=lax.Precision.HIGHEST)


def _efs_call(ea_aug, w_aug):
    return pl.pallas_call(
        _efs_body,
        grid=(EGRID,),
        in_specs=[
            pl.BlockSpec((EBLK, 24), lambda i: (i, 0)),
            pl.BlockSpec((24, 128), lambda i: (0, 0)),
        ],
        out_specs=pl.BlockSpec((EBLK, 128), lambda i: (i, 0)),
        out_shape=jax.ShapeDtypeStruct((E, 128), _f32),
    )(ea_aug, w_aug)


def _ln_relu(x, agg, g_ref, b_ref):
    y = x + agg
    mu = jnp.mean(y, axis=1, keepdims=True)
    d = y - mu
    var = jnp.mean(d * d, axis=1, keepdims=True)
    xn = d * lax.rsqrt(var + 1e-5) * g_ref[...] + b_ref[...]
    return jnp.maximum(xn, 0.0)


def _post_xform_body(x_ref, agg_ref, g_ref, b_ref, wdt_ref, wst_ref,
                     xn_ref, afs_ref, bfs_ref):
    xn = _ln_relu(x_ref[...], agg_ref[...], g_ref, b_ref)
    xn_ref[...] = xn
    afs_ref[...] = jnp.dot(xn, wdt_ref[...], preferred_element_type=_f32, precision=lax.Precision.HIGHEST)
    bfs_ref[...] = jnp.dot(xn, wst_ref[...], preferred_element_type=_f32, precision=lax.Precision.HIGHEST)


def _post_xform_call(x, agg, g, b, wdt, wst):
    return pl.pallas_call(
        _post_xform_body,
        grid=(NGRID,),
        in_specs=[
            pl.BlockSpec((NBLK, 64), lambda i: (i, 0)),
            pl.BlockSpec((NBLK, 64), lambda i: (i, 0)),
            pl.BlockSpec((1, 64), lambda i: (0, 0)),
            pl.BlockSpec((1, 64), lambda i: (0, 0)),
            pl.BlockSpec((64, 128), lambda i: (0, 0)),
            pl.BlockSpec((64, 128), lambda i: (0, 0)),
        ],
        out_specs=[
            pl.BlockSpec((NBLK, 64), lambda i: (i, 0)),
            pl.BlockSpec((NBLK, 128), lambda i: (i, 0)),
            pl.BlockSpec((NBLK, 128), lambda i: (i, 0)),
        ],
        out_shape=[
            jax.ShapeDtypeStruct((NP_, 64), _f32),
            jax.ShapeDtypeStruct((NP_, 128), _f32),
            jax.ShapeDtypeStruct((NP_, 128), _f32),
        ],
    )(x, agg, g, b, wdt, wst)


def _post_body(x_ref, agg_ref, g_ref, b_ref, xn_ref):
    xn_ref[...] = _ln_relu(x_ref[...], agg_ref[...], g_ref, b_ref)


def _post_call(x, agg, g, b):
    return pl.pallas_call(
        _post_body,
        grid=(NGRID,),
        in_specs=[
            pl.BlockSpec((NBLK, 64), lambda i: (i, 0)),
            pl.BlockSpec((NBLK, 64), lambda i: (i, 0)),
            pl.BlockSpec((1, 64), lambda i: (0, 0)),
            pl.BlockSpec((1, 64), lambda i: (0, 0)),
        ],
        out_specs=pl.BlockSpec((NBLK, 64), lambda i: (i, 0)),
        out_shape=jax.ShapeDtypeStruct((NP_, 64), _f32),
    )(x, agg, g, b)


def _pool_body(x_ref, b3_ref, w1t_ref, b1_ref, w2t_ref, b2_ref, o_ref, acc_ref):
    i = pl.program_id(0)

    @pl.when(i == 0)
    def _():
        acc_ref[...] = jnp.zeros_like(acc_ref)

    bb = b3_ref[0, 0, :]
    oh = (bb[:, None] == lax.broadcasted_iota(jnp.int32, (NBLK, NG), 1)).astype(_f32)
    xa = jnp.concatenate([x_ref[...], jnp.ones((NBLK, 64), _f32)], axis=1)
    acc_ref[...] += lax.dot_general(oh, xa, (((0,), (0,)), ((), ())),
                                    preferred_element_type=_f32, precision=lax.Precision.HIGHEST)

    @pl.when(i == NGRID - 1)
    def _():
        s = acc_ref[:, :64]
        c = acc_ref[:, 64:65]
        pooled = s / jnp.maximum(c, 1.0)
        h = jnp.maximum(jnp.dot(pooled, w1t_ref[...], preferred_element_type=_f32, precision=lax.Precision.HIGHEST)
                        + b1_ref[...], 0.0)
        o_ref[...] = jnp.dot(h, w2t_ref[...], preferred_element_type=_f32, precision=lax.Precision.HIGHEST) + b2_ref[...]


def _pool_call(x, batch3, w1t, b1, w2t, b2):
    return pl.pallas_call(
        _pool_body,
        grid=(NGRID,),
        in_specs=[
            pl.BlockSpec((NBLK, 64), lambda i: (i, 0)),
            pl.BlockSpec((1, 1, NBLK), lambda i: (i, 0, 0)),
            pl.BlockSpec((64, 32), lambda i: (0, 0)),
            pl.BlockSpec((1, 32), lambda i: (0, 0)),
            pl.BlockSpec((32, 8), lambda i: (0, 0)),
            pl.BlockSpec((1, 8), lambda i: (0, 0)),
        ],
        out_specs=pl.BlockSpec((NG, 8), lambda i: (0, 0)),
        out_shape=jax.ShapeDtypeStruct((NG, 8), _f32),
        scratch_shapes=[pltpu.VMEM((NG, 128), _f32)],
    )(x, batch3, w1t, b1, w2t, b2)


def _edge_stage(afs, bfs, efs, src, dst):
    gp = afs[dst] + bfs[src] + efs
    gate = jax.nn.sigmoid(gp[:, :64])
    core = _softplus(gp[:, 64:])
    return jax.ops.segment_sum(gate * core, dst, num_segments=NP_)


def kernel(z, x_scalar, edge_index, edge_attr, batch, atom_embed, lin0_w, lin0_b,
           convf_w, convf_b, convs_w, convs_b, ln_g, ln_b, lin1_w, lin1_b,
           lin2_w, lin2_b):
    src = edge_index[0]
    dst = edge_index[1]
    zc = jnp.clip(z, 0, MAX_Z)

    zc3 = jnp.pad(zc, (0, NP_ - N)).reshape(NGRID, 1, NBLK)
    xs_p = jnp.pad(x_scalar, ((0, NP_ - N), (0, 8 - NS)))
    ae_pad = jnp.pad(atom_embed, ((0, 128 - (MAX_Z + 2)), (0, 0)))
    waet = lin0_w[:, :HID].T
    wxs = jnp.pad(lin0_w[:, HID:].T, ((0, 8 - NS), (0, 0)))
    b0 = lin0_b[None, :]

    wdt = [jnp.concatenate([convf_w[l][:, :HID].T, convs_w[l][:, :HID].T], axis=1)
           for l in range(L)]
    wst = [jnp.concatenate([convf_w[l][:, HID:2 * HID].T,
                            convs_w[l][:, HID:2 * HID].T], axis=1) for l in range(L)]
    ea_aug = jnp.concatenate(
        [edge_attr, jnp.ones((E, 1), _f32), jnp.zeros((E, 7), _f32)], axis=1)
    we_aug = [jnp.concatenate([
        jnp.concatenate([convf_w[l][:, 2 * HID:].T, convs_w[l][:, 2 * HID:].T], axis=1),
        jnp.concatenate([convf_b[l], convs_b[l]])[None, :],
        jnp.zeros((7, 128), _f32)], axis=0) for l in range(L)]

    x, afs, bfs = _x0_call(zc3, xs_p, ae_pad, waet, b0, wxs, wdt[0], wst[0])
    efs = [_efs_call(ea_aug, we_aug[l]) for l in range(L)]

    for l in range(L):
        agg = _edge_stage(afs, bfs, efs[l], src, dst)
        if l < L - 1:
            x, afs, bfs = _post_xform_call(x, agg, ln_g[l][None, :], ln_b[l][None, :],
                                           wdt[l + 1], wst[l + 1])
        else:
            x = _post_call(x, agg, ln_g[l][None, :], ln_b[l][None, :])

    batch3 = jnp.pad(batch, (0, NP_ - N), constant_values=NG).reshape(NGRID, 1, NBLK)
    w1t = lin1_w.T
    b1 = lin1_b[None, :]
    w2t = jnp.pad(lin2_w.T, ((0, 0), (0, 7)))
    b2 = jnp.pad(lin2_b[None, :], ((0, 0), (0, 7)))
    out2 = _pool_call(x, batch3, w1t, b1, w2t, b2)
    return out2[:, 0]
